# Initial kernel scaffold; baseline (speedup 1.0000x reference)
import jax
import jax.numpy as jnp
from jax.experimental import pallas as pl


def kernel(y_true, y_pred, lookup):
    raise NotImplementedError("write your pallas kernel here")


# speedup vs baseline: 129.8402x; 129.8402x over previous
import functools

import jax
import jax.numpy as jnp
from jax import lax
from jax.experimental import pallas as pl
from jax.experimental.pallas import tpu as pltpu
from jax.experimental.pallas import tpu_sc as plsc

N = 1024
D = 64
M = 1000
ND_USE = 48
EPS = 1e-8
NPAIRS = N * (N - 1) // 2

NC = 2
NS = 16
LANES = 16
NW = NC * NS
ROWS_PER_W = N // NW

_sc_mesh = plsc.VectorSubcoreMesh(core_axis_name="c", subcore_axis_name="s")


@functools.partial(
    pl.kernel,
    mesh=_sc_mesh,
    compiler_params=pltpu.CompilerParams(
        use_tc_tiling_on_sc=False, needs_layout_passes=False),
    out_type=jax.ShapeDtypeStruct((N, N), jnp.float32),
    scratch_types=[
        pltpu.VMEM((N,), jnp.int32),
        pltpu.VMEM((ROWS_PER_W,), jnp.int32),
        pltpu.VMEM((ROWS_PER_W, M), jnp.float32),
        pltpu.VMEM((ROWS_PER_W, N), jnp.float32),
        pltpu.SemaphoreType.DMA,
    ],
)
def _sc_affinity(yt_hbm, lookup_hbm, out_hbm, yt_v, idx_v, rows_v, a_v, sem):
    wid = lax.axis_index("s") * NC + lax.axis_index("c")
    base = wid * ROWS_PER_W
    pltpu.sync_copy(yt_hbm, yt_v)
    pltpu.sync_copy(yt_hbm.at[pl.ds(base, ROWS_PER_W)], idx_v)
    pltpu.async_copy(lookup_hbm.at[idx_v], rows_v, sem).wait()

    def row_body(r, carry):
        row_splat = jnp.full((LANES,), r, jnp.int32)

        def col_body(j, carry2):
            cidx = yt_v[pl.ds(j * LANES, LANES)]
            vals = plsc.load_gather(rows_v, [row_splat, cidx])
            cpos = j * LANES + lax.iota(jnp.int32, LANES)
            plsc.store_scatter(a_v, [row_splat, cpos], vals)
            return carry2

        return lax.fori_loop(0, N // LANES, col_body, carry)

    lax.fori_loop(0, ROWS_PER_W, row_body, 0)
    pltpu.sync_copy(a_v, out_hbm.at[pl.ds(base, ROWS_PER_W)])


def _tc_loss_body(yp_ref, a_ref, out_ref):
    yp = yp_ref[:]
    col = lax.broadcasted_iota(jnp.int32, (N, D), 1)
    ypm = jnp.where(col < ND_USE, yp, 0.0)
    nrm = jnp.sqrt(jnp.sum(ypm * ypm, axis=1, keepdims=True))
    yn = ypm * (1.0 / jnp.maximum(nrm, EPS))
    g = lax.dot_general(yn, yn, (((1,), (1,)), ((), ())),
                        preferred_element_type=jnp.float32)
    ii = lax.broadcasted_iota(jnp.int32, (N, N), 0)
    jj = lax.broadcasted_iota(jnp.int32, (N, N), 1)
    diff = jnp.abs(g - a_ref[:])
    s = jnp.sum(jnp.where(jj > ii, diff, 0.0))
    out_ref[0, 0] = s * (1.0 / NPAIRS)


_tc_loss = pl.pallas_call(
    _tc_loss_body,
    out_shape=jax.ShapeDtypeStruct((1, 1), jnp.float32),
    in_specs=[
        pl.BlockSpec(memory_space=pltpu.VMEM),
        pl.BlockSpec(memory_space=pltpu.VMEM),
    ],
    out_specs=pl.BlockSpec(memory_space=pltpu.SMEM),
)


@jax.jit
def kernel(y_true, y_pred, lookup):
    yt = y_true.astype(jnp.int32)
    a = _sc_affinity(yt, lookup)
    loss = _tc_loss(y_pred, a)
    return loss[0, 0]


# speedup vs baseline: 197.1023x; 1.5180x over previous
import functools

import jax
import jax.numpy as jnp
from jax import lax
from jax.experimental import pallas as pl
from jax.experimental.pallas import tpu as pltpu
from jax.experimental.pallas import tpu_sc as plsc

N = 1024
D = 64
M = 1000
ND_USE = 48
EPS = 1e-8
NPAIRS = N * (N - 1) // 2

NC = 2
NS = 16
LANES = 16
NW = NC * NS
ROWS_PER_W = N // NW

_sc_mesh = plsc.VectorSubcoreMesh(core_axis_name="c", subcore_axis_name="s")


@functools.partial(
    pl.kernel,
    mesh=_sc_mesh,
    compiler_params=pltpu.CompilerParams(
        use_tc_tiling_on_sc=False, needs_layout_passes=False),
    out_type=jax.ShapeDtypeStruct((N, N), jnp.float32),
    scratch_types=[
        pltpu.VMEM((N,), jnp.int32),
        pltpu.VMEM((ROWS_PER_W,), jnp.int32),
        pltpu.VMEM((ROWS_PER_W, M), jnp.float32),
        pltpu.VMEM((ROWS_PER_W, N), jnp.float32),
        pltpu.SemaphoreType.DMA,
    ],
)
def _sc_affinity(yt_hbm, lookup_hbm, out_hbm, yt_v, idx_v, rows_v, a_v, sem):
    wid = lax.axis_index("s") * NC + lax.axis_index("c")
    base = wid * ROWS_PER_W
    pltpu.sync_copy(yt_hbm, yt_v)
    pltpu.sync_copy(yt_hbm.at[pl.ds(base, ROWS_PER_W)], idx_v)
    pltpu.async_copy(lookup_hbm.at[idx_v], rows_v, sem).wait()

    def col_body(j, carry):
        cidx = yt_v[pl.ds(j * LANES, LANES)]

        @plsc.parallel_loop(0, ROWS_PER_W, unroll=8)
        def row_body(r):
            row_splat = jnp.full((LANES,), r, jnp.int32)
            vals = plsc.load_gather(rows_v, [row_splat, cidx])
            a_v[r, pl.ds(j * LANES, LANES)] = vals

        return carry

    lax.fori_loop(0, N // LANES, col_body, 0)
    pltpu.sync_copy(a_v, out_hbm.at[pl.ds(base, ROWS_PER_W)])


def _tc_loss_body(yp_ref, a_ref, out_ref):
    yp = yp_ref[:]
    col = lax.broadcasted_iota(jnp.int32, (N, D), 1)
    ypm = jnp.where(col < ND_USE, yp, 0.0)
    nrm = jnp.sqrt(jnp.sum(ypm * ypm, axis=1, keepdims=True))
    yn = ypm * (1.0 / jnp.maximum(nrm, EPS))
    g = lax.dot_general(yn, yn, (((1,), (1,)), ((), ())),
                        preferred_element_type=jnp.float32)
    ii = lax.broadcasted_iota(jnp.int32, (N, N), 0)
    jj = lax.broadcasted_iota(jnp.int32, (N, N), 1)
    diff = jnp.abs(g - a_ref[:])
    s = jnp.sum(jnp.where(jj > ii, diff, 0.0))
    out_ref[0, 0] = s * (1.0 / NPAIRS)


_tc_loss = pl.pallas_call(
    _tc_loss_body,
    out_shape=jax.ShapeDtypeStruct((1, 1), jnp.float32),
    in_specs=[
        pl.BlockSpec(memory_space=pltpu.VMEM),
        pl.BlockSpec(memory_space=pltpu.VMEM),
    ],
    out_specs=pl.BlockSpec(memory_space=pltpu.SMEM),
)


@jax.jit
def kernel(y_true, y_pred, lookup):
    yt = y_true.astype(jnp.int32)
    a = _sc_affinity(yt, lookup)
    loss = _tc_loss(y_pred, a)
    return loss[0, 0]


# speedup vs baseline: 216.7457x; 1.0997x over previous
import functools

import jax
import jax.numpy as jnp
from jax import lax
from jax.experimental import pallas as pl
from jax.experimental.pallas import tpu as pltpu
from jax.experimental.pallas import tpu_sc as plsc

N = 1024
D = 64
M = 1000
ND_USE = 48
EPS = 1e-8
NPAIRS = N * (N - 1) // 2

NC = 2
NS = 16
LANES = 16
NW = NC * NS
ROWS_PER_W = N // NW

_sc_mesh = plsc.VectorSubcoreMesh(core_axis_name="c", subcore_axis_name="s")


SCR_ROWS = N * N // 128
CHUNK = ROWS_PER_W * N // 128


@functools.partial(
    pl.kernel,
    mesh=_sc_mesh,
    compiler_params=pltpu.CompilerParams(
        use_tc_tiling_on_sc=False, needs_layout_passes=False),
    out_type=jax.ShapeDtypeStruct((SCR_ROWS, 128), jnp.float32),
    scratch_types=[
        pltpu.VMEM((N,), jnp.int32),
        pltpu.VMEM((ROWS_PER_W,), jnp.int32),
        pltpu.VMEM((ROWS_PER_W, M), jnp.float32),
        pltpu.VMEM((CHUNK, 128), jnp.float32),
        pltpu.SemaphoreType.DMA,
    ],
)
def _sc_affinity(yt_hbm, lookup_hbm, out_hbm, yt_v, idx_v, rows_v, a_v, sem):
    wid = lax.axis_index("s") * NC + lax.axis_index("c")
    base = wid * ROWS_PER_W
    pltpu.sync_copy(yt_hbm, yt_v)
    pltpu.sync_copy(yt_hbm.at[pl.ds(base, ROWS_PER_W)], idx_v)
    pltpu.async_copy(lookup_hbm.at[idx_v], rows_v, sem).wait()

    def col_body(jb, carry):
        cidx = yt_v[pl.ds(jb * LANES, LANES)]
        u_base = 8 * (jb // 8)
        c_loc = LANES * (jb % 8)

        @plsc.parallel_loop(0, ROWS_PER_W, unroll=8)
        def row_body(r):
            row_splat = jnp.full((LANES,), r, jnp.int32)
            vals = plsc.load_gather(rows_v, [row_splat, cidx])
            u_loc = u_base + 64 * (r // 8) + (r % 8)
            a_v[u_loc, pl.ds(c_loc, LANES)] = vals

        return carry

    lax.fori_loop(0, N // LANES, col_body, 0)
    pltpu.sync_copy(a_v, out_hbm.at[pl.ds(wid * CHUNK, CHUNK)])


def _tc_loss_body(yp_ref, a_ref, out_ref):
    yp = yp_ref[:]
    col = lax.broadcasted_iota(jnp.int32, (N, D), 1)
    ypm = jnp.where(col < ND_USE, yp, 0.0)
    nrm = jnp.sqrt(jnp.sum(ypm * ypm, axis=1, keepdims=True))
    yn = ypm * (1.0 / jnp.maximum(nrm, EPS))
    blocks = []
    for tj in range(8):
        b = yn[128 * tj:128 * (tj + 1), :]
        m = lax.dot_general(yn, b, (((1,), (1,)), ((), ())),
                            preferred_element_type=jnp.float32)
        blocks.append(m.reshape(N // 8, 1, 8, 128))
    gscr = jnp.concatenate(blocks, axis=1).reshape(SCR_ROWS, 128)
    u = lax.broadcasted_iota(jnp.int32, (SCR_ROWS, 128), 0)
    c = lax.broadcasted_iota(jnp.int32, (SCR_ROWS, 128), 1)
    ii = 8 * (u // 64) + (u % 8)
    jj = 128 * ((u // 8) % 8) + c
    diff = jnp.abs(gscr - a_ref[:])
    s = jnp.sum(jnp.where(jj > ii, diff, 0.0))
    out_ref[0, 0] = s * (1.0 / NPAIRS)


_tc_loss = pl.pallas_call(
    _tc_loss_body,
    out_shape=jax.ShapeDtypeStruct((1, 1), jnp.float32),
    in_specs=[
        pl.BlockSpec(memory_space=pltpu.VMEM),
        pl.BlockSpec(memory_space=pltpu.VMEM),
    ],
    out_specs=pl.BlockSpec(memory_space=pltpu.SMEM),
)


@jax.jit
def kernel(y_true, y_pred, lookup):
    yt = y_true.astype(jnp.int32)
    a = _sc_affinity(yt, lookup)
    loss = _tc_loss(y_pred, a)
    return loss[0, 0]
